# Initial kernel scaffold; baseline (speedup 1.0000x reference)
import functools

import jax
import jax.numpy as jnp
from jax import lax
from jax.experimental import pallas as pl
from jax.experimental.pallas import tpu as pltpu
from jax.experimental.pallas import tpu_sc as plsc

NC = 2
NS = 16
NW = NC * NS


def _tables_body(a_ref, wet_ref, wnt_ref, t1_ref, t2_ref):
    a = a_ref[...]
    t1_ref[...] = jnp.dot(a, wet_ref[...], preferred_element_type=jnp.float32)
    t2_ref[...] = jnp.dot(a, wnt_ref[...], preferred_element_type=jnp.float32)


def _tables(a, wet, wnt):
    n, c = a.shape
    return pl.pallas_call(
        _tables_body,
        out_shape=[jax.ShapeDtypeStruct((n, c), jnp.float32)] * 2,
    )(a, wet, wnt)


def _make_gather_mul(n, e, c, k):
    epw = e // NW
    nchunk = epw // k
    mesh = plsc.VectorSubcoreMesh(core_axis_name="c", subcore_axis_name="s")

    @functools.partial(
        pl.kernel,
        out_type=jax.ShapeDtypeStruct((e, c), jnp.float32),
        mesh=mesh,
        scratch_types=[
            pltpu.VMEM((k,), jnp.int32),
            pltpu.VMEM((k,), jnp.int32),
            pltpu.VMEM((k, c), jnp.float32),
            pltpu.VMEM((k, c), jnp.float32),
            pltpu.VMEM((k, c), jnp.float32),
            pltpu.SemaphoreType.DMA,
            pltpu.SemaphoreType.DMA,
        ],
    )
    def gather_mul(t1_hbm, t2_hbm, src_hbm, tgt_hbm, z_hbm,
                   srcv, tgtv, r1, r2, zv, sem1, sem2):
        wid = lax.axis_index("s") * NC + lax.axis_index("c")
        base = wid * epw

        def chunk(j, carry):
            off = base + j * k
            pltpu.sync_copy(src_hbm.at[pl.ds(off, k)], srcv)
            pltpu.sync_copy(tgt_hbm.at[pl.ds(off, k)], tgtv)
            cp1 = pltpu.async_copy(t2_hbm.at[srcv], r2, sem1)
            cp2 = pltpu.async_copy(t1_hbm.at[tgtv], r1, sem2)
            cp1.wait()
            cp2.wait()

            def rows(i, carry2):
                b = i * 8
                for u in range(8):
                    zv[b + u] = r1[b + u] * r2[b + u]
                return carry2

            lax.fori_loop(0, k // 8, rows, 0)
            pltpu.sync_copy(zv, z_hbm.at[pl.ds(off, k)])
            return carry

        lax.fori_loop(0, nchunk, chunk, 0)

    return gather_mul


def _mid_body(z_ref, wft_ref, bf_ref, wct_ref, bc_ref, msg_ref):
    z = z_ref[...]
    g = jnp.where(z > 0, z, jnp.expm1(z))
    f = jax.nn.sigmoid(
        jnp.dot(g, wft_ref[...], preferred_element_type=jnp.float32)
        + bf_ref[...])
    cc = jax.nn.softplus(
        jnp.dot(g, wct_ref[...], preferred_element_type=jnp.float32)
        + bc_ref[...])
    msg_ref[...] = f * cc


def _middle(z, wft, bf2, wct, bc2, be):
    e, c = z.shape
    return pl.pallas_call(
        _mid_body,
        grid=(e // be,),
        in_specs=[
            pl.BlockSpec((be, c), lambda i: (i, 0)),
            pl.BlockSpec((c, c), lambda i: (0, 0)),
            pl.BlockSpec((1, c), lambda i: (0, 0)),
            pl.BlockSpec((c, c), lambda i: (0, 0)),
            pl.BlockSpec((1, c), lambda i: (0, 0)),
        ],
        out_specs=pl.BlockSpec((be, c), lambda i: (i, 0)),
        out_shape=jax.ShapeDtypeStruct((e, c), jnp.float32),
    )(z, wft, bf2, wct, bc2)


def _make_scatter(n, e, c, k):
    epw = e // NW
    nchunk = epw // k
    rows_per_tile = n // NS
    mesh = plsc.VectorSubcoreMesh(core_axis_name="c", subcore_axis_name="s")

    @functools.partial(
        pl.kernel,
        out_type=jax.ShapeDtypeStruct((NC, n, c), jnp.float32),
        mesh=mesh,
        scratch_types=[
            pltpu.VMEM((k,), jnp.int32),
            pltpu.VMEM((k, c), jnp.float32),
            pltpu.VMEM_SHARED((n, c), jnp.float32),
        ],
    )
    def scatter(msg_hbm, src_hbm, zeros_hbm, p_hbm, srcv, msgv, acc):
        core = lax.axis_index("c")
        sub = lax.axis_index("s")
        wid = sub * NC + core
        base = wid * epw
        row0 = sub * rows_per_tile

        pltpu.sync_copy(zeros_hbm.at[pl.ds(row0, rows_per_tile)],
                        acc.at[pl.ds(row0, rows_per_tile)])
        plsc.subcore_barrier()

        def chunk(j, carry):
            off = base + j * k
            pltpu.sync_copy(src_hbm.at[pl.ds(off, k)], srcv)
            pltpu.sync_copy(msg_hbm.at[pl.ds(off, k)], msgv)
            pltpu.sync_copy(msgv, acc.at[srcv], add=True)
            return carry

        lax.fori_loop(0, nchunk, chunk, 0)
        plsc.subcore_barrier()
        pltpu.sync_copy(acc.at[pl.ds(row0, rows_per_tile)],
                        p_hbm.at[core, pl.ds(row0, rows_per_tile)])

    return scatter


def _finale_body(a_ref, p_ref, g_ref, b_ref, o_ref):
    a = a_ref[...]
    tot = a + p_ref[0] + p_ref[1]
    mean = jnp.mean(tot, axis=0, keepdims=True)
    var = jnp.mean((tot - mean) ** 2, axis=0, keepdims=True)
    bn = (tot - mean) * lax.rsqrt(var + 1e-5) * g_ref[...] + b_ref[...]
    o_ref[...] = jax.nn.softplus(a + bn)


def _finale(a, p, g2, b2):
    n, c = a.shape
    return pl.pallas_call(
        _finale_body,
        out_shape=jax.ShapeDtypeStruct((n, c), jnp.float32),
    )(a, p, g2, b2)


def kernel(atom_in_fea, edge_sources, edge_targets, w_edge0, w_node0,
           w_lin, b_lin, bn_gamma, bn_beta):
    n, c = atom_in_fea.shape[0], atom_in_fea.shape[1]
    e = edge_sources.shape[0]

    a = atom_in_fea.reshape(n, c)
    wet = w_edge0[:, :, 1, 1].T
    wnt = w_node0[:, :, 1, 1].T
    wft = w_lin[:c, :, 1, 1].T
    wct = w_lin[c:, :, 1, 1].T
    bf2 = b_lin[:c].reshape(1, c)
    bc2 = b_lin[c:].reshape(1, c)
    g2 = bn_gamma.reshape(1, c)
    b2 = bn_beta.reshape(1, c)

    t1, t2 = _tables(a, wet, wnt)
    z = _make_gather_mul(n, e, c, 1000)(t1, t2, edge_sources, edge_targets)
    msg = _middle(z, wft, bf2, wct, bc2, 8000)
    zeros = jnp.zeros((n, c), jnp.float32)
    p = _make_scatter(n, e, c, 1000)(msg, edge_sources, zeros)
    out = _finale(a, p, g2, b2)
    return out.reshape(n, c, 1, 1)


# speedup vs baseline: 19.7284x; 19.7284x over previous
import functools

import jax
import jax.numpy as jnp
from jax import lax
from jax.experimental import pallas as pl
from jax.experimental.pallas import tpu as pltpu
from jax.experimental.pallas import tpu_sc as plsc

NC = 2
NS = 16
NW = NC * NS


def _tables_body(a_ref, we8_ref, wn8_ref, t1_ref, t2_ref):
    a = a_ref[...]
    t1_ref[...] = jnp.dot(a, we8_ref[...], preferred_element_type=jnp.float32)
    t2_ref[...] = jnp.dot(a, wn8_ref[...], preferred_element_type=jnp.float32)


def _tables(a8, we8, wn8):
    r = a8.shape[0]
    return pl.pallas_call(
        _tables_body,
        out_shape=[jax.ShapeDtypeStruct((r, 128), jnp.float32)] * 2,
    )(a8, we8, wn8)


def _make_gather_mul(n, e, c, k):
    epw = e // NW
    nchunk = epw // k
    mesh = plsc.VectorSubcoreMesh(core_axis_name="c", subcore_axis_name="s")

    @functools.partial(
        pl.kernel,
        out_type=jax.ShapeDtypeStruct((e, c), jnp.float32),
        mesh=mesh,
        scratch_types=[
            pltpu.VMEM((k,), jnp.int32),
            pltpu.VMEM((k,), jnp.int32),
            pltpu.VMEM((k, c), jnp.float32),
            pltpu.VMEM((k, c), jnp.float32),
            pltpu.VMEM((k, c), jnp.float32),
            pltpu.SemaphoreType.DMA,
            pltpu.SemaphoreType.DMA,
        ],
        compiler_params=pltpu.CompilerParams(use_tc_tiling_on_sc=False),
    )
    def gather_mul(t1_hbm, t2_hbm, src_hbm, tgt_hbm, z_hbm,
                   srcv, tgtv, r1, r2, zv, sem1, sem2):
        wid = lax.axis_index("s") * NC + lax.axis_index("c")
        base = wid * epw

        def chunk(j, carry):
            off = base + j * k
            pltpu.sync_copy(src_hbm.at[pl.ds(off, k)], srcv)
            pltpu.sync_copy(tgt_hbm.at[pl.ds(off, k)], tgtv)
            cp1 = pltpu.async_copy(t2_hbm.at[srcv], r2, sem1)
            cp2 = pltpu.async_copy(t1_hbm.at[tgtv], r1, sem2)
            cp1.wait()
            cp2.wait()

            def rows(i, carry2):
                b = i * 8
                for u in range(8):
                    zv[b + u] = r1[b + u] * r2[b + u]
                return carry2

            lax.fori_loop(0, k // 8, rows, 0)
            pltpu.sync_copy(zv, z_hbm.at[pl.ds(off, k)])
            return carry

        lax.fori_loop(0, nchunk, chunk, 0)

    return gather_mul


def _mid_body(z_ref, wf8_ref, bf_ref, wc8_ref, bc_ref, msg_ref):
    z = z_ref[...]
    g = jnp.where(z > 0, z, jnp.exp(jnp.minimum(z, 0.0)) - 1.0)
    f = jax.nn.sigmoid(
        jnp.dot(g, wf8_ref[...], preferred_element_type=jnp.float32)
        + bf_ref[...])
    cc = jax.nn.softplus(
        jnp.dot(g, wc8_ref[...], preferred_element_type=jnp.float32)
        + bc_ref[...])
    msg_ref[...] = f * cc


def _middle(z8, wf8, bf128, wc8, bc128, bb):
    r = z8.shape[0]
    return pl.pallas_call(
        _mid_body,
        grid=(r // bb,),
        in_specs=[
            pl.BlockSpec((bb, 128), lambda i: (i, 0)),
            pl.BlockSpec((128, 128), lambda i: (0, 0)),
            pl.BlockSpec((1, 128), lambda i: (0, 0)),
            pl.BlockSpec((128, 128), lambda i: (0, 0)),
            pl.BlockSpec((1, 128), lambda i: (0, 0)),
        ],
        out_specs=pl.BlockSpec((bb, 128), lambda i: (i, 0)),
        out_shape=jax.ShapeDtypeStruct((r, 128), jnp.float32),
    )(z8, wf8, bf128, wc8, bc128)


def _make_scatter(n, e, c, k):
    epw = e // NW
    nchunk = epw // k
    rows_per_tile = n // NS
    mesh = plsc.VectorSubcoreMesh(core_axis_name="c", subcore_axis_name="s")

    @functools.partial(
        pl.kernel,
        out_type=jax.ShapeDtypeStruct((NC, n, c), jnp.float32),
        mesh=mesh,
        scratch_types=[
            pltpu.VMEM((k,), jnp.int32),
            pltpu.VMEM((k, c), jnp.float32),
            pltpu.VMEM_SHARED((n, c), jnp.float32),
        ],
        compiler_params=pltpu.CompilerParams(use_tc_tiling_on_sc=False),
    )
    def scatter(msg_hbm, src_hbm, zeros_hbm, p_hbm, srcv, msgv, acc):
        core = lax.axis_index("c")
        sub = lax.axis_index("s")
        wid = sub * NC + core
        base = wid * epw
        row0 = sub * rows_per_tile

        pltpu.sync_copy(zeros_hbm.at[pl.ds(row0, rows_per_tile)],
                        acc.at[pl.ds(row0, rows_per_tile)])
        plsc.subcore_barrier()

        def chunk(j, carry):
            off = base + j * k
            pltpu.sync_copy(src_hbm.at[pl.ds(off, k)], srcv)
            pltpu.sync_copy(msg_hbm.at[pl.ds(off, k)], msgv)
            pltpu.sync_copy(msgv, acc.at[srcv], add=True)
            return carry

        lax.fori_loop(0, nchunk, chunk, 0)
        plsc.subcore_barrier()
        pltpu.sync_copy(acc.at[pl.ds(row0, rows_per_tile)],
                        p_hbm.at[core, pl.ds(row0, rows_per_tile)])

    return scatter


def _finale_body(n, a_ref, p_ref, g_ref, b_ref, o_ref):
    a = a_ref[...]
    tot = a + p_ref[0] + p_ref[1]
    ri = lax.broadcasted_iota(jnp.int32, (128, 128), 0)
    ci = lax.broadcasted_iota(jnp.int32, (128, 128), 1)
    sel = jnp.where((ri % 16) == (ci % 16), 1.0 / n, 0.0).astype(jnp.float32)
    s1 = jnp.sum(tot, axis=0, keepdims=True)
    mean = jnp.dot(s1, sel, preferred_element_type=jnp.float32)
    cen = tot - mean
    s2 = jnp.sum(cen * cen, axis=0, keepdims=True)
    var = jnp.dot(s2, sel, preferred_element_type=jnp.float32)
    bn = cen * lax.rsqrt(var + 1e-5) * g_ref[...] + b_ref[...]
    o_ref[...] = jax.nn.softplus(a + bn)


def _finale(n, a8, p8, g128, b128):
    r = a8.shape[0]
    return pl.pallas_call(
        functools.partial(_finale_body, n),
        out_shape=jax.ShapeDtypeStruct((r, 128), jnp.float32),
    )(a8, p8, g128, b128)


def kernel(atom_in_fea, edge_sources, edge_targets, w_edge0, w_node0,
           w_lin, b_lin, bn_gamma, bn_beta):
    n, c = atom_in_fea.shape[0], atom_in_fea.shape[1]
    e = edge_sources.shape[0]
    eye8 = jnp.eye(8, dtype=jnp.float32)

    a8 = atom_in_fea.reshape(n // 8, 8 * c)
    we8 = jnp.kron(eye8, w_edge0[:, :, 1, 1].T)
    wn8 = jnp.kron(eye8, w_node0[:, :, 1, 1].T)
    wf8 = jnp.kron(eye8, w_lin[:c, :, 1, 1].T)
    wc8 = jnp.kron(eye8, w_lin[c:, :, 1, 1].T)
    bf128 = jnp.tile(b_lin[:c], 8).reshape(1, 8 * c)
    bc128 = jnp.tile(b_lin[c:], 8).reshape(1, 8 * c)
    g128 = jnp.tile(bn_gamma, 8).reshape(1, 8 * c)
    b128 = jnp.tile(bn_beta, 8).reshape(1, 8 * c)

    t1p, t2p = _tables(a8, we8, wn8)
    t1 = t1p.reshape(n, c)
    t2 = t2p.reshape(n, c)
    z = _make_gather_mul(n, e, c, 1000)(t1, t2, edge_sources, edge_targets)
    msg8 = _middle(z.reshape(e // 8, 8 * c), wf8, bf128, wc8, bc128, 8000)
    zeros = jnp.zeros((n, c), jnp.float32)
    p = _make_scatter(n, e, c, 1000)(msg8.reshape(e, c), edge_sources, zeros)
    out = _finale(n, a8, p.reshape(NC, n // 8, 8 * c), g128, b128)
    return out.reshape(n, c, 1, 1)


# speedup vs baseline: 26.6800x; 1.3524x over previous
import functools

import jax
import jax.numpy as jnp
from jax import lax
from jax.experimental import pallas as pl
from jax.experimental.pallas import tpu as pltpu
from jax.experimental.pallas import tpu_sc as plsc

NC = 2
NS = 16
NW = NC * NS


def _tables_body(a_ref, we8_ref, wn8_ref, t1_ref, t2_ref):
    a = a_ref[...]
    t1_ref[...] = jnp.dot(a, we8_ref[...], preferred_element_type=jnp.float32)
    t2_ref[...] = jnp.dot(a, wn8_ref[...], preferred_element_type=jnp.float32)


def _tables(a8, we8, wn8):
    r = a8.shape[0]
    return pl.pallas_call(
        _tables_body,
        out_shape=[jax.ShapeDtypeStruct((r, 128), jnp.float32)] * 2,
    )(a8, we8, wn8)


def _make_gather_mul(n, e, c, k):
    epw = e // NW
    nchunk = epw // k
    mesh = plsc.VectorSubcoreMesh(core_axis_name="c", subcore_axis_name="s")

    @functools.partial(
        pl.kernel,
        out_type=jax.ShapeDtypeStruct((e, c), jnp.float32),
        mesh=mesh,
        scratch_types=[
            pltpu.VMEM((k,), jnp.int32),
            pltpu.VMEM((k,), jnp.int32),
            pltpu.VMEM((k,), jnp.int32),
            pltpu.VMEM((k,), jnp.int32),
            pltpu.VMEM((k, c), jnp.float32),
            pltpu.VMEM((k, c), jnp.float32),
            pltpu.VMEM((k, c), jnp.float32),
            pltpu.VMEM((k, c), jnp.float32),
            pltpu.VMEM((k, c), jnp.float32),
            pltpu.VMEM((k, c), jnp.float32),
            pltpu.SemaphoreType.DMA,
            pltpu.SemaphoreType.DMA,
            pltpu.SemaphoreType.DMA,
            pltpu.SemaphoreType.DMA,
        ],
        compiler_params=pltpu.CompilerParams(use_tc_tiling_on_sc=False),
    )
    def gather_mul(t1_hbm, t2_hbm, src_hbm, tgt_hbm, z_hbm,
                   srcv0, srcv1, tgtv0, tgtv1, r10, r11, r20, r21,
                   zv0, zv1, gsem0, gsem1, wsem0, wsem1):
        wid = lax.axis_index("s") * NC + lax.axis_index("c")
        base = wid * epw
        sets = ((srcv0, tgtv0, r10, r20, zv0, gsem0, wsem0),
                (srcv1, tgtv1, r11, r21, zv1, gsem1, wsem1))

        def issue(j, s):
            srcv, tgtv, r1, r2, gsem, _ = s[0], s[1], s[2], s[3], s[5], None
            off = base + j * k
            pltpu.sync_copy(src_hbm.at[pl.ds(off, k)], srcv)
            pltpu.sync_copy(tgt_hbm.at[pl.ds(off, k)], tgtv)
            pltpu.async_copy(t2_hbm.at[srcv], r2, gsem)
            pltpu.async_copy(t1_hbm.at[tgtv], r1, gsem)

        def consume(m, j, s):
            srcv, tgtv, r1, r2, zv, gsem, wsem = s
            pltpu.make_async_copy(t2_hbm.at[srcv], r2, gsem).wait()
            pltpu.make_async_copy(t1_hbm.at[tgtv], r1, gsem).wait()
            @pl.when(m > 0)
            def _():
                pltpu.make_async_copy(
                    zv, z_hbm.at[pl.ds(base, k)], wsem).wait()

            def rows(i, carry2):
                b = i * 8
                for u in range(8):
                    zv[b + u] = r1[b + u] * r2[b + u]
                return carry2

            lax.fori_loop(0, k // 8, rows, 0)
            pltpu.async_copy(zv, z_hbm.at[pl.ds(base + j * k, k)], wsem)

        issue(0, sets[0])

        def pair(m, carry):
            issue(2 * m + 1, sets[1])
            consume(m, 2 * m, sets[0])

            @pl.when(m < nchunk // 2 - 1)
            def _():
                issue(2 * m + 2, sets[0])

            consume(m, 2 * m + 1, sets[1])
            return carry

        lax.fori_loop(0, nchunk // 2, pair, 0)
        pltpu.make_async_copy(zv0, z_hbm.at[pl.ds(base, k)], wsem0).wait()
        pltpu.make_async_copy(zv1, z_hbm.at[pl.ds(base, k)], wsem1).wait()

    return gather_mul


def _mid_body(z_ref, wf8_ref, bf_ref, wc8_ref, bc_ref, msg_ref):
    z = z_ref[...]
    g = jnp.where(z > 0, z, jnp.exp(jnp.minimum(z, 0.0)) - 1.0)
    f = jax.nn.sigmoid(
        jnp.dot(g, wf8_ref[...], preferred_element_type=jnp.float32)
        + bf_ref[...])
    cc = jax.nn.softplus(
        jnp.dot(g, wc8_ref[...], preferred_element_type=jnp.float32)
        + bc_ref[...])
    msg_ref[...] = f * cc


def _middle(z8, wf8, bf128, wc8, bc128, bb):
    r = z8.shape[0]
    return pl.pallas_call(
        _mid_body,
        grid=(r // bb,),
        in_specs=[
            pl.BlockSpec((bb, 128), lambda i: (i, 0)),
            pl.BlockSpec((128, 128), lambda i: (0, 0)),
            pl.BlockSpec((1, 128), lambda i: (0, 0)),
            pl.BlockSpec((128, 128), lambda i: (0, 0)),
            pl.BlockSpec((1, 128), lambda i: (0, 0)),
        ],
        out_specs=pl.BlockSpec((bb, 128), lambda i: (i, 0)),
        out_shape=jax.ShapeDtypeStruct((r, 128), jnp.float32),
    )(z8, wf8, bf128, wc8, bc128)


def _make_scatter(n, e, c, k):
    epw = e // NW
    nchunk = epw // k
    rows_per_tile = n // NS
    mesh = plsc.VectorSubcoreMesh(core_axis_name="c", subcore_axis_name="s")

    @functools.partial(
        pl.kernel,
        out_type=jax.ShapeDtypeStruct((NC, n, c), jnp.float32),
        mesh=mesh,
        scratch_types=[
            pltpu.VMEM((k,), jnp.int32),
            pltpu.VMEM((k,), jnp.int32),
            pltpu.VMEM((k, c), jnp.float32),
            pltpu.VMEM((k, c), jnp.float32),
            pltpu.SemaphoreType.DMA,
            pltpu.SemaphoreType.DMA,
            pltpu.VMEM_SHARED((n, c), jnp.float32),
        ],
        compiler_params=pltpu.CompilerParams(use_tc_tiling_on_sc=False),
    )
    def scatter(msg_hbm, src_hbm, zeros_hbm, p_hbm,
                srcv0, srcv1, msgv0, msgv1, lsem0, lsem1, acc):
        core = lax.axis_index("c")
        sub = lax.axis_index("s")
        wid = sub * NC + core
        base = wid * epw
        row0 = sub * rows_per_tile

        pltpu.sync_copy(zeros_hbm.at[pl.ds(row0, rows_per_tile)],
                        acc.at[pl.ds(row0, rows_per_tile)])
        plsc.subcore_barrier()
        sets = ((srcv0, msgv0, lsem0), (srcv1, msgv1, lsem1))

        def issue(j, s):
            srcv, msgv, lsem = s
            off = base + j * k
            pltpu.sync_copy(src_hbm.at[pl.ds(off, k)], srcv)
            pltpu.async_copy(msg_hbm.at[pl.ds(off, k)], msgv, lsem)

        def consume(s):
            srcv, msgv, lsem = s
            pltpu.make_async_copy(
                msg_hbm.at[pl.ds(base, k)], msgv, lsem).wait()
            pltpu.sync_copy(msgv, acc.at[srcv], add=True)

        issue(0, sets[0])

        def pair(m, carry):
            issue(2 * m + 1, sets[1])
            consume(sets[0])

            @pl.when(m < nchunk // 2 - 1)
            def _():
                issue(2 * m + 2, sets[0])

            consume(sets[1])
            return carry

        lax.fori_loop(0, nchunk // 2, pair, 0)
        plsc.subcore_barrier()
        pltpu.sync_copy(acc.at[pl.ds(row0, rows_per_tile)],
                        p_hbm.at[core, pl.ds(row0, rows_per_tile)])

    return scatter


def _finale_body(n, a_ref, p_ref, g_ref, b_ref, o_ref):
    a = a_ref[...]
    tot = a + p_ref[0] + p_ref[1]
    ri = lax.broadcasted_iota(jnp.int32, (128, 128), 0)
    ci = lax.broadcasted_iota(jnp.int32, (128, 128), 1)
    sel = jnp.where((ri % 16) == (ci % 16), 1.0 / n, 0.0).astype(jnp.float32)
    s1 = jnp.sum(tot, axis=0, keepdims=True)
    mean = jnp.dot(s1, sel, preferred_element_type=jnp.float32)
    cen = tot - mean
    s2 = jnp.sum(cen * cen, axis=0, keepdims=True)
    var = jnp.dot(s2, sel, preferred_element_type=jnp.float32)
    bn = cen * lax.rsqrt(var + 1e-5) * g_ref[...] + b_ref[...]
    o_ref[...] = jax.nn.softplus(a + bn)


def _finale(n, a8, p8, g128, b128):
    r = a8.shape[0]
    return pl.pallas_call(
        functools.partial(_finale_body, n),
        out_shape=jax.ShapeDtypeStruct((r, 128), jnp.float32),
    )(a8, p8, g128, b128)


def kernel(atom_in_fea, edge_sources, edge_targets, w_edge0, w_node0,
           w_lin, b_lin, bn_gamma, bn_beta):
    n, c = atom_in_fea.shape[0], atom_in_fea.shape[1]
    e = edge_sources.shape[0]
    eye8 = jnp.eye(8, dtype=jnp.float32)

    a8 = atom_in_fea.reshape(n // 8, 8 * c)
    we8 = jnp.kron(eye8, w_edge0[:, :, 1, 1].T)
    wn8 = jnp.kron(eye8, w_node0[:, :, 1, 1].T)
    wf8 = jnp.kron(eye8, w_lin[:c, :, 1, 1].T)
    wc8 = jnp.kron(eye8, w_lin[c:, :, 1, 1].T)
    bf128 = jnp.tile(b_lin[:c], 8).reshape(1, 8 * c)
    bc128 = jnp.tile(b_lin[c:], 8).reshape(1, 8 * c)
    g128 = jnp.tile(bn_gamma, 8).reshape(1, 8 * c)
    b128 = jnp.tile(bn_beta, 8).reshape(1, 8 * c)

    t1p, t2p = _tables(a8, we8, wn8)
    t1 = t1p.reshape(n, c)
    t2 = t2p.reshape(n, c)
    z = _make_gather_mul(n, e, c, 1000)(t1, t2, edge_sources, edge_targets)
    msg8 = _middle(z.reshape(e // 8, 8 * c), wf8, bf128, wc8, bc128, 8000)
    zeros = jnp.zeros((n, c), jnp.float32)
    p = _make_scatter(n, e, c, 1000)(msg8.reshape(e, c), edge_sources, zeros)
    out = _finale(n, a8, p.reshape(NC, n // 8, 8 * c), g128, b128)
    return out.reshape(n, c, 1, 1)


# speedup vs baseline: 28.5060x; 1.0684x over previous
import functools

import jax
import jax.numpy as jnp
from jax import lax
from jax.experimental import pallas as pl
from jax.experimental.pallas import tpu as pltpu
from jax.experimental.pallas import tpu_sc as plsc

NC = 2
NS = 16
NW = NC * NS


def _tables_body(a_ref, we8_ref, wn8_ref, t1_ref, t2_ref):
    a = a_ref[...]
    t1_ref[...] = jnp.dot(a, we8_ref[...], preferred_element_type=jnp.float32)
    t2_ref[...] = jnp.dot(a, wn8_ref[...], preferred_element_type=jnp.float32)


def _tables(a8, we8, wn8):
    r = a8.shape[0]
    return pl.pallas_call(
        _tables_body,
        out_shape=[jax.ShapeDtypeStruct((r, 128), jnp.float32)] * 2,
    )(a8, we8, wn8)


def _make_gather_mul(n, e, c, k):
    epw = e // NW
    nchunk = epw // k
    mesh = plsc.VectorSubcoreMesh(core_axis_name="c", subcore_axis_name="s")

    @functools.partial(
        pl.kernel,
        out_type=jax.ShapeDtypeStruct((e, c), jnp.float32),
        mesh=mesh,
        scratch_types=[
            pltpu.VMEM((k,), jnp.int32),
            pltpu.VMEM((k,), jnp.int32),
            pltpu.VMEM((k,), jnp.int32),
            pltpu.VMEM((k,), jnp.int32),
            pltpu.VMEM((k, c), jnp.float32),
            pltpu.VMEM((k, c), jnp.float32),
            pltpu.VMEM((k, c), jnp.float32),
            pltpu.VMEM((k, c), jnp.float32),
            pltpu.VMEM((k, c), jnp.float32),
            pltpu.VMEM((k, c), jnp.float32),
            pltpu.SemaphoreType.DMA,
            pltpu.SemaphoreType.DMA,
            pltpu.SemaphoreType.DMA,
            pltpu.SemaphoreType.DMA,
        ],
        compiler_params=pltpu.CompilerParams(use_tc_tiling_on_sc=False),
    )
    def gather_mul(t1_hbm, t2_hbm, src_hbm, tgt_hbm, z_hbm,
                   srcv0, srcv1, tgtv0, tgtv1, r10, r11, r20, r21,
                   zv0, zv1, gsem0, gsem1, wsem0, wsem1):
        wid = lax.axis_index("s") * NC + lax.axis_index("c")
        base = wid * epw
        sets = ((srcv0, tgtv0, r10, r20, zv0, gsem0, wsem0),
                (srcv1, tgtv1, r11, r21, zv1, gsem1, wsem1))

        def issue(j, s):
            srcv, tgtv, r1, r2, gsem, _ = s[0], s[1], s[2], s[3], s[5], None
            off = base + j * k
            pltpu.sync_copy(src_hbm.at[pl.ds(off, k)], srcv)
            pltpu.sync_copy(tgt_hbm.at[pl.ds(off, k)], tgtv)
            pltpu.async_copy(t2_hbm.at[srcv], r2, gsem)
            pltpu.async_copy(t1_hbm.at[tgtv], r1, gsem)

        def consume(m, j, s):
            srcv, tgtv, r1, r2, zv, gsem, wsem = s
            pltpu.make_async_copy(t2_hbm.at[srcv], r2, gsem).wait()
            pltpu.make_async_copy(t1_hbm.at[tgtv], r1, gsem).wait()
            @pl.when(m > 0)
            def _():
                pltpu.make_async_copy(
                    zv, z_hbm.at[pl.ds(base, k)], wsem).wait()

            def rows(i, carry2):
                b = i * 8
                for u in range(8):
                    zv[b + u] = r1[b + u] * r2[b + u]
                return carry2

            lax.fori_loop(0, k // 8, rows, 0)
            pltpu.async_copy(zv, z_hbm.at[pl.ds(base + j * k, k)], wsem)

        issue(0, sets[0])

        def pair(m, carry):
            issue(2 * m + 1, sets[1])
            consume(m, 2 * m, sets[0])
            issue(2 * m + 2, sets[0])
            consume(m, 2 * m + 1, sets[1])
            return carry

        lax.fori_loop(0, nchunk // 2, pair, 0)
        consume(1, nchunk - 1, sets[0])
        pltpu.make_async_copy(zv0, z_hbm.at[pl.ds(base, k)], wsem0).wait()
        pltpu.make_async_copy(zv1, z_hbm.at[pl.ds(base, k)], wsem1).wait()

    return gather_mul


def _mid_body(z_ref, wf8_ref, bf_ref, wc8_ref, bc_ref, msg_ref):
    z = z_ref[...]
    g = jnp.where(z > 0, z, jnp.exp(jnp.minimum(z, 0.0)) - 1.0)
    f = jax.nn.sigmoid(
        jnp.dot(g, wf8_ref[...], preferred_element_type=jnp.float32)
        + bf_ref[...])
    cc = jax.nn.softplus(
        jnp.dot(g, wc8_ref[...], preferred_element_type=jnp.float32)
        + bc_ref[...])
    msg_ref[...] = f * cc


def _middle(z8, wf8, bf128, wc8, bc128, bb):
    r = z8.shape[0]
    return pl.pallas_call(
        _mid_body,
        grid=(r // bb,),
        in_specs=[
            pl.BlockSpec((bb, 128), lambda i: (i, 0)),
            pl.BlockSpec((128, 128), lambda i: (0, 0)),
            pl.BlockSpec((1, 128), lambda i: (0, 0)),
            pl.BlockSpec((128, 128), lambda i: (0, 0)),
            pl.BlockSpec((1, 128), lambda i: (0, 0)),
        ],
        out_specs=pl.BlockSpec((bb, 128), lambda i: (i, 0)),
        out_shape=jax.ShapeDtypeStruct((r, 128), jnp.float32),
    )(z8, wf8, bf128, wc8, bc128)


def _make_scatter(n, e, c, k):
    epw = e // NW
    nchunk = epw // k
    rows_per_tile = n // NS
    mesh = plsc.VectorSubcoreMesh(core_axis_name="c", subcore_axis_name="s")

    @functools.partial(
        pl.kernel,
        out_type=jax.ShapeDtypeStruct((NC, n, c), jnp.float32),
        mesh=mesh,
        scratch_types=[
            pltpu.VMEM((k,), jnp.int32),
            pltpu.VMEM((k,), jnp.int32),
            pltpu.VMEM((k, c), jnp.float32),
            pltpu.VMEM((k, c), jnp.float32),
            pltpu.SemaphoreType.DMA,
            pltpu.SemaphoreType.DMA,
            pltpu.VMEM_SHARED((n, c), jnp.float32),
        ],
        compiler_params=pltpu.CompilerParams(use_tc_tiling_on_sc=False),
    )
    def scatter(msg_hbm, src_hbm, zeros_hbm, p_hbm,
                srcv0, srcv1, msgv0, msgv1, lsem0, lsem1, acc):
        core = lax.axis_index("c")
        sub = lax.axis_index("s")
        wid = sub * NC + core
        base = wid * epw
        row0 = sub * rows_per_tile

        pltpu.sync_copy(zeros_hbm.at[pl.ds(row0, rows_per_tile)],
                        acc.at[pl.ds(row0, rows_per_tile)])
        plsc.subcore_barrier()
        sets = ((srcv0, msgv0, lsem0), (srcv1, msgv1, lsem1))

        def issue(j, s):
            srcv, msgv, lsem = s
            off = base + j * k
            pltpu.sync_copy(src_hbm.at[pl.ds(off, k)], srcv)
            pltpu.async_copy(msg_hbm.at[pl.ds(off, k)], msgv, lsem)

        def consume(s):
            srcv, msgv, lsem = s
            pltpu.make_async_copy(
                msg_hbm.at[pl.ds(base, k)], msgv, lsem).wait()
            pltpu.sync_copy(msgv, acc.at[srcv], add=True)

        issue(0, sets[0])

        def pair(m, carry):
            issue(2 * m + 1, sets[1])
            consume(sets[0])
            issue(2 * m + 2, sets[0])
            consume(sets[1])
            return carry

        lax.fori_loop(0, nchunk // 2, pair, 0)
        consume(sets[0])
        plsc.subcore_barrier()
        pltpu.sync_copy(acc.at[pl.ds(row0, rows_per_tile)],
                        p_hbm.at[core, pl.ds(row0, rows_per_tile)])

    return scatter


def _finale_body(n, a_ref, pa_ref, pb_ref, g_ref, b_ref, o_ref):
    a = a_ref[...]
    tot = a + (pa_ref[0] + pa_ref[1]) + (pb_ref[0] + pb_ref[1])
    ri = lax.broadcasted_iota(jnp.int32, (128, 128), 0)
    ci = lax.broadcasted_iota(jnp.int32, (128, 128), 1)
    sel = jnp.where((ri % 16) == (ci % 16), 1.0 / n, 0.0).astype(jnp.float32)
    s1 = jnp.sum(tot, axis=0, keepdims=True)
    mean = jnp.dot(s1, sel, preferred_element_type=jnp.float32)
    cen = tot - mean
    s2 = jnp.sum(cen * cen, axis=0, keepdims=True)
    var = jnp.dot(s2, sel, preferred_element_type=jnp.float32)
    bn = cen * lax.rsqrt(var + 1e-5) * g_ref[...] + b_ref[...]
    o_ref[...] = jax.nn.softplus(a + bn)


def _finale(n, a8, pa8, pb8, g128, b128):
    r = a8.shape[0]
    return pl.pallas_call(
        functools.partial(_finale_body, n),
        out_shape=jax.ShapeDtypeStruct((r, 128), jnp.float32),
    )(a8, pa8, pb8, g128, b128)


def kernel(atom_in_fea, edge_sources, edge_targets, w_edge0, w_node0,
           w_lin, b_lin, bn_gamma, bn_beta):
    n, c = atom_in_fea.shape[0], atom_in_fea.shape[1]
    e = edge_sources.shape[0]
    eye8 = jnp.eye(8, dtype=jnp.float32)

    a8 = atom_in_fea.reshape(n // 8, 8 * c)
    we8 = jnp.kron(eye8, w_edge0[:, :, 1, 1].T)
    wn8 = jnp.kron(eye8, w_node0[:, :, 1, 1].T)
    wf8 = jnp.kron(eye8, w_lin[:c, :, 1, 1].T)
    wc8 = jnp.kron(eye8, w_lin[c:, :, 1, 1].T)
    bf128 = jnp.tile(b_lin[:c], 8).reshape(1, 8 * c)
    bc128 = jnp.tile(b_lin[c:], 8).reshape(1, 8 * c)
    g128 = jnp.tile(bn_gamma, 8).reshape(1, 8 * c)
    b128 = jnp.tile(bn_beta, 8).reshape(1, 8 * c)

    t1p, t2p = _tables(a8, we8, wn8)
    t1 = t1p.reshape(n, c)
    t2 = t2p.reshape(n, c)
    zeros = jnp.zeros((n, c), jnp.float32)

    eh = e // 2
    gm = _make_gather_mul(n, eh, c, 1000)
    sc = _make_scatter(n, eh, c, 1000)
    src_a, src_b = edge_sources[:eh], edge_sources[eh:]
    tgt_a, tgt_b = edge_targets[:eh], edge_targets[eh:]
    z_a = gm(t1, t2, src_a, tgt_a)
    z_b = gm(t1, t2, src_b, tgt_b)
    msg_a = _middle(z_a.reshape(eh // 8, 8 * c), wf8, bf128, wc8, bc128, 10000)
    msg_b = _middle(z_b.reshape(eh // 8, 8 * c), wf8, bf128, wc8, bc128, 10000)
    p_a = sc(msg_a.reshape(eh, c), src_a, zeros)
    p_b = sc(msg_b.reshape(eh, c), src_b, zeros)
    out = _finale(n, a8, p_a.reshape(NC, n // 8, 8 * c),
                  p_b.reshape(NC, n // 8, 8 * c), g128, b128)
    return out.reshape(n, c, 1, 1)


# speedup vs baseline: 28.6853x; 1.0063x over previous
import functools

import jax
import jax.numpy as jnp
from jax import lax
from jax.experimental import pallas as pl
from jax.experimental.pallas import tpu as pltpu
from jax.experimental.pallas import tpu_sc as plsc

NC = 2
NS = 16
NW = NC * NS


def _tables_body(a_ref, we8_ref, wn8_ref, t1_ref, t2_ref):
    a = a_ref[...]
    t1_ref[...] = jnp.dot(a, we8_ref[...], preferred_element_type=jnp.float32)
    t2_ref[...] = jnp.dot(a, wn8_ref[...], preferred_element_type=jnp.float32)


def _tables(a8, we8, wn8):
    r = a8.shape[0]
    return pl.pallas_call(
        _tables_body,
        out_shape=[jax.ShapeDtypeStruct((r, 128), jnp.float32)] * 2,
    )(a8, we8, wn8)


def _make_gather_mul(n, e, c, k):
    epw = e // NW
    nchunk = epw // k
    mesh = plsc.VectorSubcoreMesh(core_axis_name="c", subcore_axis_name="s")

    @functools.partial(
        pl.kernel,
        out_type=jax.ShapeDtypeStruct((e, c), jnp.float32),
        mesh=mesh,
        scratch_types=[
            pltpu.VMEM((k,), jnp.int32),
            pltpu.VMEM((k,), jnp.int32),
            pltpu.VMEM((k,), jnp.int32),
            pltpu.VMEM((k,), jnp.int32),
            pltpu.VMEM((k, c), jnp.float32),
            pltpu.VMEM((k, c), jnp.float32),
            pltpu.VMEM((k, c), jnp.float32),
            pltpu.VMEM((k, c), jnp.float32),
            pltpu.VMEM((k, c), jnp.float32),
            pltpu.VMEM((k, c), jnp.float32),
            pltpu.SemaphoreType.DMA,
            pltpu.SemaphoreType.DMA,
            pltpu.SemaphoreType.DMA,
            pltpu.SemaphoreType.DMA,
            pltpu.SemaphoreType.DMA,
            pltpu.SemaphoreType.DMA,
        ],
        compiler_params=pltpu.CompilerParams(use_tc_tiling_on_sc=False),
    )
    def gather_mul(t1_hbm, t2_hbm, src_hbm, tgt_hbm, z_hbm,
                   srcv0, srcv1, tgtv0, tgtv1, r10, r11, r20, r21,
                   zv0, zv1, isem0, isem1, gsem0, gsem1, wsem0, wsem1):
        wid = lax.axis_index("s") * NC + lax.axis_index("c")
        base = wid * epw
        sets = ((srcv0, tgtv0, r10, r20, zv0, isem0, gsem0, wsem0),
                (srcv1, tgtv1, r11, r21, zv1, isem1, gsem1, wsem1))

        def issue_idx(j, s):
            srcv, tgtv, isem = s[0], s[1], s[5]
            off = base + j * k
            pltpu.async_copy(src_hbm.at[pl.ds(off, k)], srcv, isem)
            pltpu.async_copy(tgt_hbm.at[pl.ds(off, k)], tgtv, isem)

        def start_gather(s):
            srcv, tgtv, r1, r2, isem, gsem = s[0], s[1], s[2], s[3], s[5], s[6]
            pltpu.make_async_copy(src_hbm.at[pl.ds(base, k)], srcv, isem).wait()
            pltpu.make_async_copy(tgt_hbm.at[pl.ds(base, k)], tgtv, isem).wait()
            pltpu.async_copy(t2_hbm.at[srcv], r2, gsem)
            pltpu.async_copy(t1_hbm.at[tgtv], r1, gsem)

        def wait_gather(s):
            srcv, tgtv, r1, r2, gsem = s[0], s[1], s[2], s[3], s[6]
            pltpu.make_async_copy(t2_hbm.at[srcv], r2, gsem).wait()
            pltpu.make_async_copy(t1_hbm.at[tgtv], r1, gsem).wait()

        def compute(first, j, s):
            r1, r2, zv, wsem = s[2], s[3], s[4], s[7]

            @pl.when(jnp.logical_not(first))
            def _():
                pltpu.make_async_copy(
                    zv, z_hbm.at[pl.ds(base, k)], wsem).wait()

            def rows(i, carry2):
                b = i * 8
                for u in range(8):
                    zv[b + u] = r1[b + u] * r2[b + u]
                return carry2

            lax.fori_loop(0, k // 8, rows, 0)
            pltpu.async_copy(zv, z_hbm.at[pl.ds(base + j * k, k)], wsem)

        p_half = nchunk // 2
        issue_idx(0, sets[0])
        start_gather(sets[0])
        issue_idx(1, sets[1])

        def pair(m, carry):
            wait_gather(sets[0])
            issue_idx(2 * m + 2, sets[0])
            start_gather(sets[1])
            compute(m == 0, 2 * m, sets[0])
            wait_gather(sets[1])

            @pl.when(m < p_half - 1)
            def _():
                issue_idx(2 * m + 3, sets[1])

            start_gather(sets[0])
            compute(m == 0, 2 * m + 1, sets[1])
            return carry

        lax.fori_loop(0, p_half, pair, 0)
        wait_gather(sets[0])
        compute(False, nchunk - 1, sets[0])
        pltpu.make_async_copy(zv0, z_hbm.at[pl.ds(base, k)], wsem0).wait()
        pltpu.make_async_copy(zv1, z_hbm.at[pl.ds(base, k)], wsem1).wait()

    return gather_mul


def _mid_body(z_ref, wf8_ref, bf_ref, wc8_ref, bc_ref, msg_ref):
    z = z_ref[...]
    g = jnp.where(z > 0, z, jnp.exp(jnp.minimum(z, 0.0)) - 1.0)
    f = jax.nn.sigmoid(
        jnp.dot(g, wf8_ref[...], preferred_element_type=jnp.float32)
        + bf_ref[...])
    cc = jax.nn.softplus(
        jnp.dot(g, wc8_ref[...], preferred_element_type=jnp.float32)
        + bc_ref[...])
    msg_ref[...] = f * cc


def _middle(z8, wf8, bf128, wc8, bc128, bb):
    r = z8.shape[0]
    return pl.pallas_call(
        _mid_body,
        grid=(r // bb,),
        in_specs=[
            pl.BlockSpec((bb, 128), lambda i: (i, 0)),
            pl.BlockSpec((128, 128), lambda i: (0, 0)),
            pl.BlockSpec((1, 128), lambda i: (0, 0)),
            pl.BlockSpec((128, 128), lambda i: (0, 0)),
            pl.BlockSpec((1, 128), lambda i: (0, 0)),
        ],
        out_specs=pl.BlockSpec((bb, 128), lambda i: (i, 0)),
        out_shape=jax.ShapeDtypeStruct((r, 128), jnp.float32),
    )(z8, wf8, bf128, wc8, bc128)


def _make_scatter(n, e, c, k):
    epw = e // NW
    nchunk = epw // k
    rows_per_tile = n // NS
    mesh = plsc.VectorSubcoreMesh(core_axis_name="c", subcore_axis_name="s")

    @functools.partial(
        pl.kernel,
        out_type=jax.ShapeDtypeStruct((NC, n, c), jnp.float32),
        mesh=mesh,
        scratch_types=[
            pltpu.VMEM((k,), jnp.int32),
            pltpu.VMEM((k,), jnp.int32),
            pltpu.VMEM((k, c), jnp.float32),
            pltpu.VMEM((k, c), jnp.float32),
            pltpu.SemaphoreType.DMA,
            pltpu.SemaphoreType.DMA,
            pltpu.VMEM_SHARED((n, c), jnp.float32),
        ],
        compiler_params=pltpu.CompilerParams(use_tc_tiling_on_sc=False),
    )
    def scatter(msg_hbm, src_hbm, zeros_hbm, p_hbm,
                srcv0, srcv1, msgv0, msgv1, lsem0, lsem1, acc):
        core = lax.axis_index("c")
        sub = lax.axis_index("s")
        wid = sub * NC + core
        base = wid * epw
        row0 = sub * rows_per_tile

        pltpu.sync_copy(zeros_hbm.at[pl.ds(row0, rows_per_tile)],
                        acc.at[pl.ds(row0, rows_per_tile)])
        plsc.subcore_barrier()
        sets = ((srcv0, msgv0, lsem0), (srcv1, msgv1, lsem1))

        def issue(j, s):
            srcv, msgv, lsem = s
            off = base + j * k
            pltpu.async_copy(src_hbm.at[pl.ds(off, k)], srcv, lsem)
            pltpu.async_copy(msg_hbm.at[pl.ds(off, k)], msgv, lsem)

        def consume(s):
            srcv, msgv, lsem = s
            pltpu.make_async_copy(
                src_hbm.at[pl.ds(base, k)], srcv, lsem).wait()
            pltpu.make_async_copy(
                msg_hbm.at[pl.ds(base, k)], msgv, lsem).wait()
            pltpu.sync_copy(msgv, acc.at[srcv], add=True)

        issue(0, sets[0])

        def pair(m, carry):
            issue(2 * m + 1, sets[1])
            consume(sets[0])
            issue(2 * m + 2, sets[0])
            consume(sets[1])
            return carry

        lax.fori_loop(0, nchunk // 2, pair, 0)
        consume(sets[0])
        plsc.subcore_barrier()
        pltpu.sync_copy(acc.at[pl.ds(row0, rows_per_tile)],
                        p_hbm.at[core, pl.ds(row0, rows_per_tile)])

    return scatter


def _finale_body(n, a_ref, pa_ref, pb_ref, g_ref, b_ref, o_ref):
    a = a_ref[...]
    tot = a + (pa_ref[0] + pa_ref[1]) + (pb_ref[0] + pb_ref[1])
    ri = lax.broadcasted_iota(jnp.int32, (128, 128), 0)
    ci = lax.broadcasted_iota(jnp.int32, (128, 128), 1)
    sel = jnp.where((ri % 16) == (ci % 16), 1.0 / n, 0.0).astype(jnp.float32)
    s1 = jnp.sum(tot, axis=0, keepdims=True)
    mean = jnp.dot(s1, sel, preferred_element_type=jnp.float32)
    cen = tot - mean
    s2 = jnp.sum(cen * cen, axis=0, keepdims=True)
    var = jnp.dot(s2, sel, preferred_element_type=jnp.float32)
    bn = cen * lax.rsqrt(var + 1e-5) * g_ref[...] + b_ref[...]
    o_ref[...] = jax.nn.softplus(a + bn)


def _finale(n, a8, pa8, pb8, g128, b128):
    r = a8.shape[0]
    return pl.pallas_call(
        functools.partial(_finale_body, n),
        out_shape=jax.ShapeDtypeStruct((r, 128), jnp.float32),
    )(a8, pa8, pb8, g128, b128)


def kernel(atom_in_fea, edge_sources, edge_targets, w_edge0, w_node0,
           w_lin, b_lin, bn_gamma, bn_beta):
    n, c = atom_in_fea.shape[0], atom_in_fea.shape[1]
    e = edge_sources.shape[0]
    eye8 = jnp.eye(8, dtype=jnp.float32)

    a8 = atom_in_fea.reshape(n // 8, 8 * c)
    we8 = jnp.kron(eye8, w_edge0[:, :, 1, 1].T)
    wn8 = jnp.kron(eye8, w_node0[:, :, 1, 1].T)
    wf8 = jnp.kron(eye8, w_lin[:c, :, 1, 1].T)
    wc8 = jnp.kron(eye8, w_lin[c:, :, 1, 1].T)
    bf128 = jnp.tile(b_lin[:c], 8).reshape(1, 8 * c)
    bc128 = jnp.tile(b_lin[c:], 8).reshape(1, 8 * c)
    g128 = jnp.tile(bn_gamma, 8).reshape(1, 8 * c)
    b128 = jnp.tile(bn_beta, 8).reshape(1, 8 * c)

    t1p, t2p = _tables(a8, we8, wn8)
    t1 = t1p.reshape(n, c)
    t2 = t2p.reshape(n, c)
    zeros = jnp.zeros((n, c), jnp.float32)

    eh = e // 2
    gm = _make_gather_mul(n, eh, c, 1000)
    sc = _make_scatter(n, eh, c, 1000)
    src_a, src_b = edge_sources[:eh], edge_sources[eh:]
    tgt_a, tgt_b = edge_targets[:eh], edge_targets[eh:]
    z_a = gm(t1, t2, src_a, tgt_a)
    z_b = gm(t1, t2, src_b, tgt_b)
    msg_a = _middle(z_a.reshape(eh // 8, 8 * c), wf8, bf128, wc8, bc128, 10000)
    msg_b = _middle(z_b.reshape(eh // 8, 8 * c), wf8, bf128, wc8, bc128, 10000)
    p_a = sc(msg_a.reshape(eh, c), src_a, zeros)
    p_b = sc(msg_b.reshape(eh, c), src_b, zeros)
    out = _finale(n, a8, p_a.reshape(NC, n // 8, 8 * c),
                  p_b.reshape(NC, n // 8, 8 * c), g128, b128)
    return out.reshape(n, c, 1, 1)
